# baseline (speedup 1.0000x reference)
import functools

import jax
import jax.numpy as jnp
from jax import lax
from jax.experimental import pallas as pl
from jax.experimental.pallas import tpu as pltpu
from jax.experimental.pallas import tpu_sc as plsc

_USER = 25000
_ITEM = 25000
_N = _USER + _ITEM
_NP = 51200
_DH = 32
_E = 800000
_NC, _NS = 2, 16
_CHUNK = 256
_NCH = 198
_RING = 6
_NRINGS = _NCH // _RING
_EPT = _CHUNK * _NCH
_E_PAD = _EPT * _NS
_RPT = _NP // _NS


@functools.partial(
    pl.kernel,
    out_type=[
        jax.ShapeDtypeStruct((2 * _NP, _DH), jnp.float32),
        jax.ShapeDtypeStruct((2 * _NP, _DH), jnp.float32),
        jax.ShapeDtypeStruct((2 * _NP, _DH), jnp.bfloat16),
        jax.ShapeDtypeStruct((2 * _NP, _DH), jnp.bfloat16),
        jax.ShapeDtypeStruct((2 * _NP, _DH), jnp.bfloat16),
        jax.ShapeDtypeStruct((_NP, 2 * _DH), jnp.float32),
        jax.ShapeDtypeStruct((_NP, 2 * _DH), jnp.float32),
    ],
    mesh=plsc.VectorSubcoreMesh(
        core_axis_name="c", subcore_axis_name="s",
        num_cores=_NC, num_subcores=_NS,
    ),
    compiler_params=pltpu.CompilerParams(
        use_tc_tiling_on_sc=False, needs_layout_passes=False),
    scratch_types=(
        [pltpu.VMEM((3, _CHUNK), jnp.int32)] * 3
        + [pltpu.VMEM((_CHUNK, _DH), jnp.bfloat16)] * 2
        + [pltpu.VMEM((_CHUNK, _DH), jnp.float32)] * 2
        + [pltpu.VMEM((_CHUNK,), jnp.int32)] * 2
        + [pltpu.VMEM_SHARED((_NP, _DH), jnp.float32)]
        + [pltpu.SemaphoreType.DMA] * 7
    ),
)
def _sc_prop(ego0, meta_h, zeros_h, t1, t2, t0b, t1b, t2b, e3, mean_out, *scr):
    meta = scr[0:3]
    rbf = scr[3:5]
    rf32 = scr[5:7]
    dstv = scr[7:9]
    acc = scr[9]
    sem_m = scr[10:13]
    sem_g = scr[13:15]
    sem_s = scr[15:17]

    cc = lax.axis_index("c")
    s = lax.axis_index("s")
    half = pl.multiple_of(cc * _NP, 8)
    r0 = pl.multiple_of(s * _RPT, 8)
    ch0 = s * _NCH

    def meta_start(idx, m):
        pltpu.async_copy(meta_h.at[idx], meta[m], sem_m[m])

    def meta_wait(m):
        pltpu.make_async_copy(meta_h.at[0], meta[m], sem_m[m]).wait()

    def src_offset(m):
        @pl.when(cc == 1)
        def _():
            for j in range(_CHUNK // 16):
                meta[m][0, pl.ds(j * 16, 16)] = \
                    meta[m][0, pl.ds(j * 16, 16)] + _NP

    def gather_start(tinb, p, m):
        src_offset(m)
        pltpu.async_copy(tinb.at[meta[m].at[0]], rbf[p], sem_g[p])

    def gather_wait(tinb, p, m):
        pltpu.make_async_copy(tinb.at[meta[m].at[0]], rbf[p], sem_g[p]).wait()

    def scatter_start(p):
        pltpu.make_async_copy(rf32[p], acc.at[dstv[p]], sem_s[p]).start(add=True)

    def scatter_wait(p):
        pltpu.make_async_copy(rf32[p], acc.at[dstv[p]], sem_s[p]).wait()

    def dst_copy(p, m):
        for j in range(_CHUNK // 16):
            dstv[p][pl.ds(j * 16, 16)] = meta[m][1, pl.ds(j * 16, 16)]

    def scale(p, m):
        def _grp(grp, cy):
            g16 = plsc.bitcast(meta[m][2, pl.ds(grp * 16, 16)], jnp.float32)
            e_base = grp * 16
            for j in range(16):
                g = g16[j]
                e = e_base + j
                ab = rbf[p][e, pl.ds(0, _DH)]
                a, b = plsc.unpack(ab, format=plsc.PackFormat.INTERLEAVED)
                rf32[p][e, pl.ds(0, 16)] = a * g
                rf32[p][e, pl.ds(16, 16)] = b * g
            return cy
        lax.fori_loop(0, _CHUNK // 16, _grp, 0)

    def pack_rows(size):
        def _pk(e, cy):
            a = rf32[0][e, pl.ds(0, 16)]
            b = rf32[0][e, pl.ds(16, 16)]
            rbf[0][e, pl.ds(0, _DH)] = plsc.pack(
                a, b, format=plsc.PackFormat.INTERLEAVED)
            return cy
        lax.fori_loop(0, size, _pk, 0)

    def stage_steps(fn):
        def _w(i, carry):
            fn(r0 + i * _CHUNK, _CHUNK)
            return carry
        lax.fori_loop(0, _RPT // _CHUNK, _w, 0)
        rem = _RPT % _CHUNK
        if rem:
            fn(r0 + (_RPT // _CHUNK) * _CHUNK, rem)

    def run_layer(tinb, write_fn):
        pltpu.sync_copy(zeros_h, acc.at[pl.ds(r0, _RPT)])
        plsc.subcore_barrier()

        for m in range(3):
            meta_start(ch0 + m, m)
        meta_wait(0)
        gather_start(tinb, 0, 0)

        def _ring(k, carry):
            for j in range(_RING):
                c = ch0 + _RING * k + j
                p, m = j % 2, j % 3
                q, m1 = (j + 1) % 2, (j + 1) % 3
                gather_wait(tinb, p, m)
                if j < 2:
                    @pl.when(k >= 1)
                    def _():
                        scatter_wait(p)
                else:
                    scatter_wait(p)
                dst_copy(p, m)
                if j == _RING - 1:
                    @pl.when(k < _NRINGS - 1)
                    def _():
                        meta_wait(m1)
                        gather_start(tinb, q, m1)
                else:
                    meta_wait(m1)
                    gather_start(tinb, q, m1)
                scale(p, m)
                scatter_start(p)
                if j < 3:
                    meta_start(c + 3, m)
                else:
                    @pl.when(k < _NRINGS - 1)
                    def _():
                        meta_start(c + 3, m)
            return carry
        lax.fori_loop(0, _NRINGS, _ring, 0)
        scatter_wait(0)
        scatter_wait(1)
        plsc.subcore_barrier()

        write_fn()
        plsc.subcore_barrier()

    def pack_table(src_f32_hbm, dst_bf_hbm):
        def _step(rr, size):
            pltpu.sync_copy(src_f32_hbm.at[pl.ds(half + rr, size)],
                            rf32[0].at[pl.ds(0, size)])
            pack_rows(size)
            pltpu.sync_copy(rbf[0].at[pl.ds(0, size)],
                            dst_bf_hbm.at[pl.ds(half + rr, size)])
        stage_steps(_step)

    def wb_plain(tout, toutb):
        def _step(rr, size):
            pltpu.sync_copy(acc.at[pl.ds(rr, size)],
                            rf32[0].at[pl.ds(0, size)])
            pltpu.sync_copy(rf32[0].at[pl.ds(0, size)],
                            tout.at[pl.ds(half + rr, size)])
            pack_rows(size)
            pltpu.sync_copy(rbf[0].at[pl.ds(0, size)],
                            toutb.at[pl.ds(half + rr, size)])
        stage_steps(_step)

    def wb_final():
        col = cc * _DH
        pltpu.sync_copy(acc.at[pl.ds(r0, _RPT)],
                        e3.at[pl.ds(r0, _RPT), pl.ds(col, _DH)])
        b1, b2 = rf32[0], rf32[1]

        def _step(rr, size):
            pltpu.sync_copy(t1.at[pl.ds(half + rr, size)],
                            b1.at[pl.ds(0, size)])
            pltpu.sync_copy(t2.at[pl.ds(half + rr, size)],
                            b2.at[pl.ds(0, size)])

            def _m1(rI, cy):
                b1[rI, pl.ds(0, 16)] = b1[rI, pl.ds(0, 16)] \
                    + b2[rI, pl.ds(0, 16)]
                b1[rI, pl.ds(16, 16)] = b1[rI, pl.ds(16, 16)] \
                    + b2[rI, pl.ds(16, 16)]
                return cy
            lax.fori_loop(0, size, _m1, 0)
            pltpu.sync_copy(acc.at[pl.ds(rr, size)], b2.at[pl.ds(0, size)])

            def _m2(rI, cy):
                b1[rI, pl.ds(0, 16)] = (b1[rI, pl.ds(0, 16)] * 2.0
                                        + b2[rI, pl.ds(0, 16)]) * 0.2
                b1[rI, pl.ds(16, 16)] = (b1[rI, pl.ds(16, 16)] * 2.0
                                         + b2[rI, pl.ds(16, 16)]) * 0.2
                return cy
            lax.fori_loop(0, size, _m2, 0)
            pltpu.sync_copy(b1.at[pl.ds(0, size)],
                            mean_out.at[pl.ds(rr, size), pl.ds(col, _DH)])
        stage_steps(_step)

    pack_table(ego0, t0b)
    plsc.subcore_barrier()

    run_layer(t0b, lambda: wb_plain(t1, t1b))
    run_layer(t1b, lambda: wb_plain(t2, t2b))
    run_layer(t2b, wb_final)


def kernel(user_emb, item_emb, adj_values, edge_index):
    ego = jnp.concatenate([user_emb, item_emb], axis=0)
    rpad = jnp.zeros((_NP - _N, _DH), jnp.float32)
    ego0 = jnp.concatenate([ego[:, :_DH], rpad, ego[:, _DH:], rpad], axis=0)
    src = edge_index[0].astype(jnp.int32)
    dst = edge_index[1].astype(jnp.int32)
    gain = adj_values.astype(jnp.float32)
    pad = _E_PAD - _E
    pidx = (jnp.arange(pad, dtype=jnp.int32) * 61) % _N
    srcp = jnp.concatenate([src, pidx])
    dstp = jnp.concatenate([dst, pidx])
    gbits = lax.bitcast_convert_type(
        jnp.concatenate([gain, jnp.zeros((pad,), jnp.float32)]), jnp.int32)
    meta = jnp.stack([srcp.reshape(_NS * _NCH, _CHUNK),
                      dstp.reshape(_NS * _NCH, _CHUNK),
                      gbits.reshape(_NS * _NCH, _CHUNK)], axis=1)
    zeros_h = jnp.zeros((_RPT, _DH), jnp.float32)

    outs = _sc_prop(ego0, meta, zeros_h)
    e3, mean = outs[5], outs[6]

    return (mean[:_USER], mean[_USER:_N], e3[_USER:_N])


# speedup vs baseline: 1.5698x; 1.5698x over previous
import functools

import jax
import jax.numpy as jnp
from jax import lax
from jax.experimental import pallas as pl
from jax.experimental.pallas import tpu as pltpu
from jax.experimental.pallas import tpu_sc as plsc

_USER = 25000
_ITEM = 25000
_N = _USER + _ITEM
_NP = 51200
_DH = 32
_E = 800000
_NC, _NS = 2, 16
_CHUNK = 192
_NCH = 264
_NSETS = 4
_NRINGS = _NCH // _NSETS
_EPT = _CHUNK * _NCH
_E_PAD = _EPT * _NS
_RPT = _NP // _NS
_WB = 128
_WB_STEPS = _RPT // _WB


@functools.partial(
    pl.kernel,
    out_type=[
        jax.ShapeDtypeStruct((2 * _NP, _DH), jnp.float32),
        jax.ShapeDtypeStruct((2 * _NP, _DH), jnp.float32),
        jax.ShapeDtypeStruct((_NP, 2 * _DH), jnp.float32),
        jax.ShapeDtypeStruct((_NP, 2 * _DH), jnp.float32),
    ],
    mesh=plsc.VectorSubcoreMesh(
        core_axis_name="c", subcore_axis_name="s",
        num_cores=_NC, num_subcores=_NS,
    ),
    compiler_params=pltpu.CompilerParams(
        use_tc_tiling_on_sc=False, needs_layout_passes=False),
    scratch_types=(
        [pltpu.VMEM((3, _CHUNK), jnp.int32)] * _NSETS
        + [pltpu.VMEM((_CHUNK, _DH), jnp.float32)] * _NSETS
        + [pltpu.VMEM((_CHUNK,), jnp.int32)] * _NSETS
        + [pltpu.VMEM_SHARED((_NP, _DH), jnp.float32)]
        + [pltpu.SemaphoreType.DMA] * (3 * _NSETS)
    ),
)
def _sc_prop(ego0, meta_h, zeros_h, t1, t2, e3, mean_out, *scr):
    meta = scr[0:_NSETS]
    rows = scr[_NSETS:2 * _NSETS]
    dstv = scr[2 * _NSETS:3 * _NSETS]
    acc = scr[3 * _NSETS]
    sem_m = scr[3 * _NSETS + 1:4 * _NSETS + 1]
    sem_g = scr[4 * _NSETS + 1:5 * _NSETS + 1]
    sem_s = scr[5 * _NSETS + 1:6 * _NSETS + 1]

    cc = lax.axis_index("c")
    s = lax.axis_index("s")
    half = pl.multiple_of(cc * _NP, 8)
    r0 = pl.multiple_of(s * _RPT, 8)
    ch0 = s * _NCH

    def meta_start(idx, p):
        pltpu.async_copy(meta_h.at[idx], meta[p], sem_m[p])

    def meta_wait(p):
        pltpu.make_async_copy(meta_h.at[0], meta[p], sem_m[p]).wait()

    def src_offset(p):
        @pl.when(cc == 1)
        def _():
            for j in range(_CHUNK // 16):
                meta[p][0, pl.ds(j * 16, 16)] = \
                    meta[p][0, pl.ds(j * 16, 16)] + _NP

    def gather_start(tin, p):
        src_offset(p)
        pltpu.async_copy(tin.at[meta[p].at[0]], rows[p], sem_g[p])

    def gather_wait(tin, p):
        pltpu.make_async_copy(tin.at[meta[p].at[0]], rows[p], sem_g[p]).wait()

    def scatter_start(p):
        pltpu.make_async_copy(rows[p], acc.at[dstv[p]], sem_s[p]).start(add=True)

    def scatter_wait(p):
        pltpu.make_async_copy(rows[p], acc.at[dstv[p]], sem_s[p]).wait()

    def dst_copy(p):
        for j in range(_CHUNK // 16):
            dstv[p][pl.ds(j * 16, 16)] = meta[p][1, pl.ds(j * 16, 16)]

    def scale(p):
        def _grp(grp, cy):
            g16 = plsc.bitcast(meta[p][2, pl.ds(grp * 16, 16)], jnp.float32)
            e_base = grp * 16
            for j in range(16):
                g = g16[j]
                e = e_base + j
                rows[p][e, pl.ds(0, 16)] = rows[p][e, pl.ds(0, 16)] * g
                rows[p][e, pl.ds(16, 16)] = rows[p][e, pl.ds(16, 16)] * g
            return cy
        lax.fori_loop(0, _CHUNK // 16, _grp, 0)

    def run_layer(tin, write_fn):
        pltpu.sync_copy(zeros_h, acc.at[pl.ds(r0, _RPT)])
        plsc.subcore_barrier()

        for m in range(_NSETS):
            meta_start(ch0 + m, m)
        meta_wait(0)
        gather_start(tin, 0)
        meta_wait(1)
        gather_start(tin, 1)

        def _ring(k, carry):
            for p in range(_NSETS):
                c = ch0 + _NSETS * k + p
                q = (p + 2) % _NSETS
                gather_wait(tin, p)
                dst_copy(p)
                if p < 2:
                    @pl.when(k >= 1)
                    def _():
                        scatter_wait(q)
                else:
                    scatter_wait(q)
                if p >= 2:
                    @pl.when(k < _NRINGS - 1)
                    def _():
                        meta_wait(q)
                        gather_start(tin, q)
                else:
                    meta_wait(q)
                    gather_start(tin, q)
                scale(p)
                scatter_start(p)
                @pl.when(k < _NRINGS - 1)
                def _():
                    meta_start(c + _NSETS, p)
            return carry
        lax.fori_loop(0, _NRINGS, _ring, 0)
        scatter_wait(2)
        scatter_wait(3)
        plsc.subcore_barrier()

        write_fn()
        plsc.subcore_barrier()

    def wb_plain(tout):
        pltpu.sync_copy(acc.at[pl.ds(r0, _RPT)], tout.at[pl.ds(half + r0, _RPT)])

    def wb_final():
        col = cc * _DH
        pltpu.sync_copy(acc.at[pl.ds(r0, _RPT)],
                        e3.at[pl.ds(r0, _RPT), pl.ds(col, _DH)])
        b1, b2, b3 = rows[0], rows[1], rows[2]

        def _mean_step(rr, size):
            pltpu.sync_copy(t1.at[pl.ds(half + rr, size)],
                            b1.at[pl.ds(0, size)])
            pltpu.sync_copy(t2.at[pl.ds(half + rr, size)],
                            b2.at[pl.ds(0, size)])
            pltpu.sync_copy(acc.at[pl.ds(rr, size)], b3.at[pl.ds(0, size)])

            def _m(rI, cy):
                x0 = (b1[rI, pl.ds(0, 16)] + b2[rI, pl.ds(0, 16)]) * 2.0 \
                    + b3[rI, pl.ds(0, 16)]
                b1[rI, pl.ds(0, 16)] = x0 * 0.2
                x1 = (b1[rI, pl.ds(16, 16)] + b2[rI, pl.ds(16, 16)]) * 2.0 \
                    + b3[rI, pl.ds(16, 16)]
                b1[rI, pl.ds(16, 16)] = x1 * 0.2
                return cy
            lax.fori_loop(0, size, _m, 0)
            pltpu.sync_copy(b1.at[pl.ds(0, size)],
                            mean_out.at[pl.ds(rr, size), pl.ds(col, _DH)])

        def _w(i, carry):
            _mean_step(r0 + i * _CHUNK, _CHUNK)
            return carry
        lax.fori_loop(0, _RPT // _CHUNK, _w, 0)
        rem = _RPT % _CHUNK
        if rem:
            _mean_step(r0 + (_RPT // _CHUNK) * _CHUNK, rem)

    run_layer(ego0, lambda: wb_plain(t1))
    run_layer(t1, lambda: wb_plain(t2))
    run_layer(t2, wb_final)


def kernel(user_emb, item_emb, adj_values, edge_index):
    ego = jnp.concatenate([user_emb, item_emb], axis=0)
    rpad = jnp.zeros((_NP - _N, _DH), jnp.float32)
    ego0 = jnp.concatenate([ego[:, :_DH], rpad, ego[:, _DH:], rpad], axis=0)
    src = edge_index[0].astype(jnp.int32)
    dst = edge_index[1].astype(jnp.int32)
    gain = adj_values.astype(jnp.float32)
    pad = _E_PAD - _E
    pidx = (jnp.arange(pad, dtype=jnp.int32) * 61) % _N
    srcp = jnp.concatenate([src, pidx])
    dstp = jnp.concatenate([dst, pidx])
    gbits = lax.bitcast_convert_type(
        jnp.concatenate([gain, jnp.zeros((pad,), jnp.float32)]), jnp.int32)
    meta = jnp.stack([srcp.reshape(_NS * _NCH, _CHUNK),
                      dstp.reshape(_NS * _NCH, _CHUNK),
                      gbits.reshape(_NS * _NCH, _CHUNK)], axis=1)
    zeros_h = jnp.zeros((_RPT, _DH), jnp.float32)

    t1, t2, e3, mean = _sc_prop(ego0, meta, zeros_h)

    return (mean[:_USER], mean[_USER:_N], e3[_USER:_N])
